# Initial kernel scaffold; baseline (speedup 1.0000x reference)
import jax
import jax.numpy as jnp
from jax.experimental import pallas as pl


def kernel(h, edge_index, edge_weight, edge_attr, data, W0, b0, Ws, bs, W1, b1, W2, b2, Wroot, bconv):
    raise NotImplementedError("write your pallas kernel here")


# speedup vs baseline: 3.0085x; 3.0085x over previous
import functools

import jax
import jax.numpy as jnp
from jax import lax
from jax.experimental import pallas as pl
from jax.experimental.pallas import tpu as pltpu
from jax.experimental.pallas import tpu_sc as plsc

NC = 2
NS = 16
NW = NC * NS
CH = 1000


_SC_PARAMS = pltpu.CompilerParams(use_tc_tiling_on_sc=False)


def _mesh():
    return plsc.VectorSubcoreMesh(core_axis_name="c", subcore_axis_name="s",
                                  num_cores=NC, num_subcores=NS)


def _sc_gather(table, idx):
    n, f = table.shape
    e = idx.shape[0]
    per_w = e // NW
    nch = per_w // CH

    @functools.partial(
        pl.kernel,
        out_type=jax.ShapeDtypeStruct((e, f), jnp.float32),
        mesh=_mesh(),
        compiler_params=_SC_PARAMS,
        scratch_types=[
            pltpu.VMEM((CH,), jnp.int32),
            pltpu.VMEM((CH, f), jnp.float32),
            pltpu.SemaphoreType.DMA,
        ],
    )
    def gk(table_hbm, idx_hbm, out_hbm, idx_v, rows_v, sem):
        wid = lax.axis_index("s") * NC + lax.axis_index("c")
        base = wid * per_w
        for k in range(nch):
            off = base + k * CH
            pltpu.sync_copy(idx_hbm.at[pl.ds(off, CH)], idx_v)
            pltpu.async_copy(table_hbm.at[idx_v], rows_v, sem).wait()
            pltpu.sync_copy(rows_v, out_hbm.at[pl.ds(off, CH)])

    return gk(table, idx)


def _sc_scatter_add(vals, idx, n):
    e, f = vals.shape
    per_w = e // NW
    nch = per_w // CH
    rows_t = n // NS
    zeros = jnp.zeros((n, f), jnp.float32)

    @functools.partial(
        pl.kernel,
        out_type=jax.ShapeDtypeStruct((NC, n, f), jnp.float32),
        mesh=_mesh(),
        compiler_params=_SC_PARAMS,
        scratch_types=[
            pltpu.VMEM((CH,), jnp.int32),
            pltpu.VMEM((CH, f), jnp.float32),
            pltpu.VMEM_SHARED((n, f), jnp.float32),
            pltpu.SemaphoreType.DMA,
        ],
    )
    def sk(vals_hbm, idx_hbm, zeros_hbm, out_hbm, idx_v, rows_v, acc_sh, sem):
        c = lax.axis_index("c")
        s = lax.axis_index("s")
        wid = s * NC + c
        pltpu.sync_copy(zeros_hbm.at[pl.ds(s * rows_t, rows_t)],
                        acc_sh.at[pl.ds(s * rows_t, rows_t)])
        plsc.subcore_barrier()
        base = wid * per_w
        for k in range(nch):
            off = base + k * CH
            pltpu.sync_copy(idx_hbm.at[pl.ds(off, CH)], idx_v)
            pltpu.sync_copy(vals_hbm.at[pl.ds(off, CH)], rows_v)
            pltpu.sync_copy(rows_v, acc_sh.at[idx_v], add=True)
        plsc.subcore_barrier()
        pltpu.sync_copy(acc_sh.at[pl.ds(s * rows_t, rows_t)],
                        out_hbm.at[c, pl.ds(s * rows_t, rows_t)])

    return sk(vals, idx, zeros)


def _sc_degree(idx, n, f):
    e = idx.shape[0]
    per_w = e // NW
    nch = per_w // CH
    rows_t = n // NS
    zeros = jnp.zeros((n, f), jnp.float32)
    ones = jnp.ones((CH, f), jnp.float32)

    @functools.partial(
        pl.kernel,
        out_type=jax.ShapeDtypeStruct((NC, n, f), jnp.float32),
        mesh=_mesh(),
        compiler_params=_SC_PARAMS,
        scratch_types=[
            pltpu.VMEM((CH,), jnp.int32),
            pltpu.VMEM((CH, f), jnp.float32),
            pltpu.VMEM_SHARED((n, f), jnp.float32),
        ],
    )
    def dk(idx_hbm, zeros_hbm, ones_hbm, out_hbm, idx_v, ones_v, acc_sh):
        c = lax.axis_index("c")
        s = lax.axis_index("s")
        wid = s * NC + c
        pltpu.sync_copy(ones_hbm, ones_v)
        pltpu.sync_copy(zeros_hbm.at[pl.ds(s * rows_t, rows_t)],
                        acc_sh.at[pl.ds(s * rows_t, rows_t)])
        plsc.subcore_barrier()
        base = wid * per_w
        for k in range(nch):
            pltpu.sync_copy(idx_hbm.at[pl.ds(base + k * CH, CH)], idx_v)
            pltpu.sync_copy(ones_v, acc_sh.at[idx_v], add=True)
        plsc.subcore_barrier()
        pltpu.sync_copy(acc_sh.at[pl.ds(s * rows_t, rows_t)],
                        out_hbm.at[c, pl.ds(s * rows_t, rows_t)])

    return dk(idx, zeros, ones)


def _tc_node_proj(h, w0, b0):
    n, d = h.shape
    f = w0.shape[1]
    bn = 2000

    def body(h_ref, w_ref, b_ref, o_ref):
        acc = jnp.dot(h_ref[...], w_ref[...], preferred_element_type=jnp.float32)
        o_ref[...] = jnp.maximum(acc + b_ref[...], 0.0)

    return pl.pallas_call(
        body,
        grid=(n // bn,),
        in_specs=[
            pl.BlockSpec((bn, d), lambda i: (i, 0)),
            pl.BlockSpec((d, f), lambda i: (0, 0)),
            pl.BlockSpec((1, f), lambda i: (0, 0)),
        ],
        out_specs=pl.BlockSpec((bn, f), lambda i: (i, 0)),
        out_shape=jax.ShapeDtypeStruct((n, f), jnp.float32),
    )(h, w0, b0.reshape(1, f))


def _tc_edge_mlp(edge_attr, ws, bs, w1, b1):
    e, ng = edge_attr.shape
    k3 = ws.shape[1]
    hid = w1.shape[1]
    be = 2000

    def body(a_ref, ws_ref, bs_ref, w1_ref, b1_ref, o_ref):
        ea = jnp.dot(a_ref[...], ws_ref[...], preferred_element_type=jnp.float32)
        ea = jnp.maximum(ea + bs_ref[...], 0.0)
        t = jnp.dot(ea, w1_ref[...], preferred_element_type=jnp.float32)
        o_ref[...] = jnp.maximum(t + b1_ref[...], 0.0)

    return pl.pallas_call(
        body,
        grid=(e // be,),
        in_specs=[
            pl.BlockSpec((be, ng), lambda i: (i, 0)),
            pl.BlockSpec((ng, k3), lambda i: (0, 0)),
            pl.BlockSpec((1, k3), lambda i: (0, 0)),
            pl.BlockSpec((k3, hid), lambda i: (0, 0)),
            pl.BlockSpec((1, hid), lambda i: (0, 0)),
        ],
        out_specs=pl.BlockSpec((be, hid), lambda i: (i, 0)),
        out_shape=jax.ShapeDtypeStruct((e, hid), jnp.float32),
    )(edge_attr, ws, bs.reshape(1, k3), w1, b1.reshape(1, hid))


def _tc_msg(t, xg, w2, b2, rmat):
    e, hid = t.shape
    nf = xg.shape[1]
    kk = nf * nf
    be = 2000

    def body(t_ref, x_ref, w2_ref, b2_ref, r_ref, o_ref):
        we = jnp.dot(t_ref[...], w2_ref[...], preferred_element_type=jnp.float32)
        we = we + b2_ref[...]
        xr = jnp.dot(x_ref[...], r_ref[...], preferred_element_type=jnp.float32)
        m = xr * we
        w = kk
        while w > nf:
            w //= 2
            m = m[:, :w] + m[:, w:]
        o_ref[...] = m

    return pl.pallas_call(
        body,
        grid=(e // be,),
        in_specs=[
            pl.BlockSpec((be, hid), lambda i: (i, 0)),
            pl.BlockSpec((be, nf), lambda i: (i, 0)),
            pl.BlockSpec((hid, kk), lambda i: (0, 0)),
            pl.BlockSpec((1, kk), lambda i: (0, 0)),
            pl.BlockSpec((nf, kk), lambda i: (0, 0)),
        ],
        out_specs=pl.BlockSpec((be, nf), lambda i: (i, 0)),
        out_shape=jax.ShapeDtypeStruct((e, nf), jnp.float32),
    )(t, xg, w2, b2.reshape(1, kk), rmat)


def _tc_update(agg2, deg2, out_prev, wroot, bconv):
    _, n, f = agg2.shape
    fd = deg2.shape[2]
    bn = 2000

    def body(a_ref, d_ref, o_ref, w_ref, b_ref, out_ref):
        agg = a_ref[0] + a_ref[1]
        deg = d_ref[0, :, 0:1] + d_ref[1, :, 0:1]
        deg = jnp.maximum(deg, 1.0)
        base = jnp.dot(o_ref[...], w_ref[...], preferred_element_type=jnp.float32)
        out_ref[...] = jnp.maximum(agg / deg + base + b_ref[...], 0.0)

    return pl.pallas_call(
        body,
        grid=(n // bn,),
        in_specs=[
            pl.BlockSpec((2, bn, f), lambda i: (0, i, 0)),
            pl.BlockSpec((2, bn, fd), lambda i: (0, i, 0)),
            pl.BlockSpec((bn, f), lambda i: (i, 0)),
            pl.BlockSpec((f, f), lambda i: (0, 0)),
            pl.BlockSpec((1, f), lambda i: (0, 0)),
        ],
        out_specs=pl.BlockSpec((bn, f), lambda i: (i, 0)),
        out_shape=jax.ShapeDtypeStruct((n, f), jnp.float32),
    )(agg2, deg2, out_prev, wroot, bconv.reshape(1, f))


def kernel(h, edge_index, edge_weight, edge_attr, data,
           W0, b0, Ws, bs, W1, b1, W2, b2, Wroot, bconv):
    n = h.shape[0]
    nf = W0.shape[1]
    src = edge_index[0].astype(jnp.int32)
    dst = edge_index[1].astype(jnp.int32)

    rmat = jnp.repeat(jnp.eye(nf, dtype=jnp.float32), nf, axis=1)

    deg2 = _sc_degree(dst, n, 16)
    out = _tc_node_proj(h, W0, b0)
    t = _tc_edge_mlp(edge_attr, Ws, bs, W1, b1)

    for _ in range(2):
        xg = _sc_gather(out, src)
        msg = _tc_msg(t, xg, W2, b2, rmat)
        agg2 = _sc_scatter_add(msg, dst, n)
        out = _tc_update(agg2, deg2, out, Wroot, bconv)
    return out


# speedup vs baseline: 3.0603x; 1.0172x over previous
import functools

import jax
import jax.numpy as jnp
from jax import lax
from jax.experimental import pallas as pl
from jax.experimental.pallas import tpu as pltpu
from jax.experimental.pallas import tpu_sc as plsc

NC = 2
NS = 16
NW = NC * NS
CH = 1000


_SC_PARAMS = pltpu.CompilerParams(use_tc_tiling_on_sc=False)


def _mesh():
    return plsc.VectorSubcoreMesh(core_axis_name="c", subcore_axis_name="s",
                                  num_cores=NC, num_subcores=NS)


def _sc_gather(table, idx):
    n, f = table.shape
    e = idx.shape[0]
    per_w = e // NW
    nch = per_w // CH

    @functools.partial(
        pl.kernel,
        out_type=jax.ShapeDtypeStruct((e, f), jnp.float32),
        mesh=_mesh(),
        compiler_params=_SC_PARAMS,
        scratch_types=[
            pltpu.VMEM((CH,), jnp.int32),
            pltpu.VMEM((CH, f), jnp.float32),
            pltpu.SemaphoreType.DMA,
        ],
    )
    def gk(table_hbm, idx_hbm, out_hbm, idx_v, rows_v, sem):
        wid = lax.axis_index("s") * NC + lax.axis_index("c")
        base = wid * per_w
        for k in range(nch):
            off = base + k * CH
            pltpu.sync_copy(idx_hbm.at[pl.ds(off, CH)], idx_v)
            pltpu.async_copy(table_hbm.at[idx_v], rows_v, sem).wait()
            pltpu.sync_copy(rows_v, out_hbm.at[pl.ds(off, CH)])

    return gk(table, idx)


def _sc_scatter_add(vals, idx, n):
    e, f = vals.shape
    per_w = e // NW
    nch = per_w // CH
    rows_t = n // NS
    zeros = jnp.zeros((n, f), jnp.float32)

    @functools.partial(
        pl.kernel,
        out_type=jax.ShapeDtypeStruct((NC, n, f), jnp.float32),
        mesh=_mesh(),
        compiler_params=_SC_PARAMS,
        scratch_types=[
            pltpu.VMEM((CH,), jnp.int32),
            pltpu.VMEM((CH, f), jnp.float32),
            pltpu.VMEM_SHARED((n, f), jnp.float32),
            pltpu.SemaphoreType.DMA,
        ],
    )
    def sk(vals_hbm, idx_hbm, zeros_hbm, out_hbm, idx_v, rows_v, acc_sh, sem):
        c = lax.axis_index("c")
        s = lax.axis_index("s")
        wid = s * NC + c
        pltpu.sync_copy(zeros_hbm.at[pl.ds(s * rows_t, rows_t)],
                        acc_sh.at[pl.ds(s * rows_t, rows_t)])
        plsc.subcore_barrier()
        base = wid * per_w
        for k in range(nch):
            off = base + k * CH
            pltpu.sync_copy(idx_hbm.at[pl.ds(off, CH)], idx_v)
            pltpu.sync_copy(vals_hbm.at[pl.ds(off, CH)], rows_v)
            pltpu.sync_copy(rows_v, acc_sh.at[idx_v], add=True)
        plsc.subcore_barrier()
        pltpu.sync_copy(acc_sh.at[pl.ds(s * rows_t, rows_t)],
                        out_hbm.at[c, pl.ds(s * rows_t, rows_t)])

    return sk(vals, idx, zeros)


def _sc_degree(idx, n, f):
    e = idx.shape[0]
    per_w = e // NW
    nch = per_w // CH
    rows_t = n // NS
    zeros = jnp.zeros((n, f), jnp.float32)
    ones = jnp.ones((CH, f), jnp.float32)

    @functools.partial(
        pl.kernel,
        out_type=jax.ShapeDtypeStruct((NC, n, f), jnp.float32),
        mesh=_mesh(),
        compiler_params=_SC_PARAMS,
        scratch_types=[
            pltpu.VMEM((CH,), jnp.int32),
            pltpu.VMEM((CH, f), jnp.float32),
            pltpu.VMEM_SHARED((n, f), jnp.float32),
        ],
    )
    def dk(idx_hbm, zeros_hbm, ones_hbm, out_hbm, idx_v, ones_v, acc_sh):
        c = lax.axis_index("c")
        s = lax.axis_index("s")
        wid = s * NC + c
        pltpu.sync_copy(ones_hbm, ones_v)
        pltpu.sync_copy(zeros_hbm.at[pl.ds(s * rows_t, rows_t)],
                        acc_sh.at[pl.ds(s * rows_t, rows_t)])
        plsc.subcore_barrier()
        base = wid * per_w
        for k in range(nch):
            pltpu.sync_copy(idx_hbm.at[pl.ds(base + k * CH, CH)], idx_v)
            pltpu.sync_copy(ones_v, acc_sh.at[idx_v], add=True)
        plsc.subcore_barrier()
        pltpu.sync_copy(acc_sh.at[pl.ds(s * rows_t, rows_t)],
                        out_hbm.at[c, pl.ds(s * rows_t, rows_t)])

    return dk(idx, zeros, ones)


def _tc_node_proj(h, w0, b0):
    n, d = h.shape
    f = w0.shape[1]
    bn = 2000

    def body(h_ref, w_ref, b_ref, o_ref):
        acc = jnp.dot(h_ref[...], w_ref[...], preferred_element_type=jnp.float32)
        o_ref[...] = jnp.maximum(acc + b_ref[...], 0.0)

    return pl.pallas_call(
        body,
        grid=(n // bn,),
        in_specs=[
            pl.BlockSpec((bn, d), lambda i: (i, 0)),
            pl.BlockSpec((d, f), lambda i: (0, 0)),
            pl.BlockSpec((1, f), lambda i: (0, 0)),
        ],
        out_specs=pl.BlockSpec((bn, f), lambda i: (i, 0)),
        out_shape=jax.ShapeDtypeStruct((n, f), jnp.float32),
    )(h, w0, b0.reshape(1, f))


def _tc_edge_mlp(edge_attr, ws, bs, w1, b1):
    e, ng = edge_attr.shape
    k3 = ws.shape[1]
    hid = w1.shape[1]
    be = 2000

    def body(a_ref, ws_ref, bs_ref, w1_ref, b1_ref, o_ref):
        ea = jnp.dot(a_ref[...], ws_ref[...], preferred_element_type=jnp.float32)
        ea = jnp.maximum(ea + bs_ref[...], 0.0)
        t = jnp.dot(ea, w1_ref[...], preferred_element_type=jnp.float32)
        o_ref[...] = jnp.maximum(t + b1_ref[...], 0.0).astype(jnp.bfloat16)

    return pl.pallas_call(
        body,
        grid=(e // be,),
        in_specs=[
            pl.BlockSpec((be, ng), lambda i: (i, 0)),
            pl.BlockSpec((ng, k3), lambda i: (0, 0)),
            pl.BlockSpec((1, k3), lambda i: (0, 0)),
            pl.BlockSpec((k3, hid), lambda i: (0, 0)),
            pl.BlockSpec((1, hid), lambda i: (0, 0)),
        ],
        out_specs=pl.BlockSpec((be, hid), lambda i: (i, 0)),
        out_shape=jax.ShapeDtypeStruct((e, hid), jnp.bfloat16),
    )(edge_attr, ws, bs.reshape(1, k3), w1, b1.reshape(1, hid))


def _tc_msg(t, xg, w2, b2, rmat):
    e, hid = t.shape
    nf = xg.shape[1]
    kk = nf * nf
    be = 2000

    def body(t_ref, x_ref, w2_ref, b2_ref, r_ref, o_ref):
        we = jnp.dot(t_ref[...], w2_ref[...], preferred_element_type=jnp.float32)
        we = we + b2_ref[...]
        xb = x_ref[...].astype(jnp.bfloat16)
        xr = jnp.dot(xb, r_ref[...], preferred_element_type=jnp.float32)
        m = xr * we
        w = kk
        while w > nf:
            w //= 2
            m = m[:, :w] + m[:, w:]
        o_ref[...] = m

    return pl.pallas_call(
        body,
        grid=(e // be,),
        in_specs=[
            pl.BlockSpec((be, hid), lambda i: (i, 0)),
            pl.BlockSpec((be, nf), lambda i: (i, 0)),
            pl.BlockSpec((hid, kk), lambda i: (0, 0)),
            pl.BlockSpec((1, kk), lambda i: (0, 0)),
            pl.BlockSpec((nf, kk), lambda i: (0, 0)),
        ],
        out_specs=pl.BlockSpec((be, nf), lambda i: (i, 0)),
        out_shape=jax.ShapeDtypeStruct((e, nf), jnp.float32),
    )(t, xg, w2, b2.reshape(1, kk), rmat)


def _tc_update(agg2, deg2, out_prev, wroot, bconv):
    _, n, f = agg2.shape
    fd = deg2.shape[2]
    bn = 2000

    def body(a_ref, d_ref, o_ref, w_ref, b_ref, out_ref):
        agg = a_ref[0] + a_ref[1]
        deg = d_ref[0, :, 0:1] + d_ref[1, :, 0:1]
        deg = jnp.maximum(deg, 1.0)
        base = jnp.dot(o_ref[...], w_ref[...], preferred_element_type=jnp.float32)
        out_ref[...] = jnp.maximum(agg / deg + base + b_ref[...], 0.0)

    return pl.pallas_call(
        body,
        grid=(n // bn,),
        in_specs=[
            pl.BlockSpec((2, bn, f), lambda i: (0, i, 0)),
            pl.BlockSpec((2, bn, fd), lambda i: (0, i, 0)),
            pl.BlockSpec((bn, f), lambda i: (i, 0)),
            pl.BlockSpec((f, f), lambda i: (0, 0)),
            pl.BlockSpec((1, f), lambda i: (0, 0)),
        ],
        out_specs=pl.BlockSpec((bn, f), lambda i: (i, 0)),
        out_shape=jax.ShapeDtypeStruct((n, f), jnp.float32),
    )(agg2, deg2, out_prev, wroot, bconv.reshape(1, f))


def kernel(h, edge_index, edge_weight, edge_attr, data,
           W0, b0, Ws, bs, W1, b1, W2, b2, Wroot, bconv):
    n = h.shape[0]
    nf = W0.shape[1]
    src = edge_index[0].astype(jnp.int32)
    dst = edge_index[1].astype(jnp.int32)

    rmat = jnp.repeat(jnp.eye(nf, dtype=jnp.bfloat16), nf, axis=1)
    w2b = W2.astype(jnp.bfloat16)

    deg2 = _sc_degree(dst, n, 16)
    out = _tc_node_proj(h, W0, b0)
    t = _tc_edge_mlp(edge_attr, Ws, bs, W1, b1)

    for _ in range(2):
        xg = _sc_gather(out, src)
        msg = _tc_msg(t, xg, w2b, b2, rmat)
        agg2 = _sc_scatter_add(msg, dst, n)
        out = _tc_update(agg2, deg2, out, Wroot, bconv)
    return out


# speedup vs baseline: 3.2265x; 1.0543x over previous
import functools

import jax
import jax.numpy as jnp
from jax import lax
from jax.experimental import pallas as pl
from jax.experimental.pallas import tpu as pltpu
from jax.experimental.pallas import tpu_sc as plsc

NC = 2
NS = 16
NW = NC * NS
CH = 1000


_SC_PARAMS = pltpu.CompilerParams(use_tc_tiling_on_sc=False)


def _mesh():
    return plsc.VectorSubcoreMesh(core_axis_name="c", subcore_axis_name="s",
                                  num_cores=NC, num_subcores=NS)


def _sc_gather(table, idx):
    n, f = table.shape
    e = idx.shape[0]
    per_w = e // NW
    nch = per_w // CH

    @functools.partial(
        pl.kernel,
        out_type=jax.ShapeDtypeStruct((e, f), jnp.float32),
        mesh=_mesh(),
        compiler_params=_SC_PARAMS,
        scratch_types=[
            pltpu.VMEM((CH,), jnp.int32),
            pltpu.VMEM((CH, f), jnp.float32),
            pltpu.SemaphoreType.DMA,
        ],
    )
    def gk(table_hbm, idx_hbm, out_hbm, idx_v, rows_v, sem):
        wid = lax.axis_index("s") * NC + lax.axis_index("c")
        base = wid * per_w
        for k in range(nch):
            off = base + k * CH
            pltpu.sync_copy(idx_hbm.at[pl.ds(off, CH)], idx_v)
            pltpu.async_copy(table_hbm.at[idx_v], rows_v, sem).wait()
            pltpu.sync_copy(rows_v, out_hbm.at[pl.ds(off, CH)])

    return gk(table, idx)


def _sc_scatter_add(vals, idx, n):
    e, f = vals.shape
    per_w = e // NW
    nch = per_w // CH
    rows_t = n // NS
    zeros = jnp.zeros((n, f), jnp.float32)

    @functools.partial(
        pl.kernel,
        out_type=jax.ShapeDtypeStruct((NC, n, f), jnp.float32),
        mesh=_mesh(),
        compiler_params=_SC_PARAMS,
        scratch_types=[
            pltpu.VMEM((CH,), jnp.int32),
            pltpu.VMEM((CH, f), jnp.float32),
            pltpu.VMEM_SHARED((n, f), jnp.float32),
            pltpu.SemaphoreType.DMA,
        ],
    )
    def sk(vals_hbm, idx_hbm, zeros_hbm, out_hbm, idx_v, rows_v, acc_sh, sem):
        c = lax.axis_index("c")
        s = lax.axis_index("s")
        wid = s * NC + c
        pltpu.sync_copy(zeros_hbm.at[pl.ds(s * rows_t, rows_t)],
                        acc_sh.at[pl.ds(s * rows_t, rows_t)])
        plsc.subcore_barrier()
        base = wid * per_w
        for k in range(nch):
            off = base + k * CH
            pltpu.sync_copy(idx_hbm.at[pl.ds(off, CH)], idx_v)
            pltpu.sync_copy(vals_hbm.at[pl.ds(off, CH)], rows_v)
            pltpu.sync_copy(rows_v, acc_sh.at[idx_v], add=True)
        plsc.subcore_barrier()
        pltpu.sync_copy(acc_sh.at[pl.ds(s * rows_t, rows_t)],
                        out_hbm.at[c, pl.ds(s * rows_t, rows_t)])

    return sk(vals, idx, zeros)


def _sc_degree(idx, n, f):
    e = idx.shape[0]
    per_w = e // NW
    nch = per_w // CH
    rows_t = n // NS
    zeros = jnp.zeros((n, f), jnp.float32)
    ones = jnp.ones((CH, f), jnp.float32)

    @functools.partial(
        pl.kernel,
        out_type=jax.ShapeDtypeStruct((NC, n, f), jnp.float32),
        mesh=_mesh(),
        compiler_params=_SC_PARAMS,
        scratch_types=[
            pltpu.VMEM((CH,), jnp.int32),
            pltpu.VMEM((CH, f), jnp.float32),
            pltpu.VMEM_SHARED((n, f), jnp.float32),
        ],
    )
    def dk(idx_hbm, zeros_hbm, ones_hbm, out_hbm, idx_v, ones_v, acc_sh):
        c = lax.axis_index("c")
        s = lax.axis_index("s")
        wid = s * NC + c
        pltpu.sync_copy(ones_hbm, ones_v)
        pltpu.sync_copy(zeros_hbm.at[pl.ds(s * rows_t, rows_t)],
                        acc_sh.at[pl.ds(s * rows_t, rows_t)])
        plsc.subcore_barrier()
        base = wid * per_w
        for k in range(nch):
            pltpu.sync_copy(idx_hbm.at[pl.ds(base + k * CH, CH)], idx_v)
            pltpu.sync_copy(ones_v, acc_sh.at[idx_v], add=True)
        plsc.subcore_barrier()
        pltpu.sync_copy(acc_sh.at[pl.ds(s * rows_t, rows_t)],
                        out_hbm.at[c, pl.ds(s * rows_t, rows_t)])

    return dk(idx, zeros, ones)


def _tc_node_proj(h, w0, b0):
    n, d = h.shape
    f = w0.shape[1]
    bn = 2000

    def body(h_ref, w_ref, b_ref, o_ref):
        acc = jnp.dot(h_ref[...], w_ref[...], preferred_element_type=jnp.float32)
        o_ref[...] = jnp.maximum(acc + b_ref[...], 0.0)

    return pl.pallas_call(
        body,
        grid=(n // bn,),
        in_specs=[
            pl.BlockSpec((bn, d), lambda i: (i, 0)),
            pl.BlockSpec((d, f), lambda i: (0, 0)),
            pl.BlockSpec((1, f), lambda i: (0, 0)),
        ],
        out_specs=pl.BlockSpec((bn, f), lambda i: (i, 0)),
        out_shape=jax.ShapeDtypeStruct((n, f), jnp.float32),
    )(h, w0, b0.reshape(1, f))


def _tc_edge_mlp(edge_attr, ws, bs, w1, b1):
    e, ng = edge_attr.shape
    k3 = ws.shape[1]
    hid = w1.shape[1]
    be = 2000

    def body(a_ref, ws_ref, bs_ref, w1_ref, b1_ref, o_ref):
        ea = jnp.dot(a_ref[...], ws_ref[...], preferred_element_type=jnp.float32)
        ea = jnp.maximum(ea + bs_ref[...], 0.0)
        t = jnp.dot(ea, w1_ref[...], preferred_element_type=jnp.float32)
        o_ref[...] = jnp.maximum(t + b1_ref[...], 0.0).astype(jnp.bfloat16)

    return pl.pallas_call(
        body,
        grid=(e // be,),
        in_specs=[
            pl.BlockSpec((be, ng), lambda i: (i, 0)),
            pl.BlockSpec((ng, k3), lambda i: (0, 0)),
            pl.BlockSpec((1, k3), lambda i: (0, 0)),
            pl.BlockSpec((k3, hid), lambda i: (0, 0)),
            pl.BlockSpec((1, hid), lambda i: (0, 0)),
        ],
        out_specs=pl.BlockSpec((be, hid), lambda i: (i, 0)),
        out_shape=jax.ShapeDtypeStruct((e, hid), jnp.bfloat16),
    )(edge_attr, ws, bs.reshape(1, k3), w1, b1.reshape(1, hid))


def _tc_msg(t, xg, w2, b2, rmat):
    e, hid = t.shape
    nf = xg.shape[1]
    kk = nf * nf
    be = 4000

    def body(t_ref, x_ref, w2_ref, b2_ref, g_ref, o_ref):
        we = jnp.dot(t_ref[...], w2_ref[...], preferred_element_type=jnp.float32)
        we = (we + b2_ref[...]).astype(jnp.bfloat16)
        xt = jnp.tile(x_ref[...].astype(jnp.bfloat16), (1, nf))
        m = xt * we
        o_ref[...] = jnp.dot(m, g_ref[...], preferred_element_type=jnp.float32)

    return pl.pallas_call(
        body,
        grid=(e // be,),
        in_specs=[
            pl.BlockSpec((be, hid), lambda i: (i, 0)),
            pl.BlockSpec((be, nf), lambda i: (i, 0)),
            pl.BlockSpec((hid, kk), lambda i: (0, 0)),
            pl.BlockSpec((1, kk), lambda i: (0, 0)),
            pl.BlockSpec((kk, nf), lambda i: (0, 0)),
        ],
        out_specs=pl.BlockSpec((be, nf), lambda i: (i, 0)),
        out_shape=jax.ShapeDtypeStruct((e, nf), jnp.float32),
    )(t, xg, w2, b2.reshape(1, kk), rmat)


def _tc_update(agg2, deg2, out_prev, wroot, bconv):
    _, n, f = agg2.shape
    fd = deg2.shape[2]
    bn = 2000

    def body(a_ref, d_ref, o_ref, w_ref, b_ref, out_ref):
        agg = a_ref[0] + a_ref[1]
        deg = d_ref[0, :, 0:1] + d_ref[1, :, 0:1]
        deg = jnp.maximum(deg, 1.0)
        base = jnp.dot(o_ref[...], w_ref[...], preferred_element_type=jnp.float32)
        out_ref[...] = jnp.maximum(agg / deg + base + b_ref[...], 0.0)

    return pl.pallas_call(
        body,
        grid=(n // bn,),
        in_specs=[
            pl.BlockSpec((2, bn, f), lambda i: (0, i, 0)),
            pl.BlockSpec((2, bn, fd), lambda i: (0, i, 0)),
            pl.BlockSpec((bn, f), lambda i: (i, 0)),
            pl.BlockSpec((f, f), lambda i: (0, 0)),
            pl.BlockSpec((1, f), lambda i: (0, 0)),
        ],
        out_specs=pl.BlockSpec((bn, f), lambda i: (i, 0)),
        out_shape=jax.ShapeDtypeStruct((n, f), jnp.float32),
    )(agg2, deg2, out_prev, wroot, bconv.reshape(1, f))


def kernel(h, edge_index, edge_weight, edge_attr, data,
           W0, b0, Ws, bs, W1, b1, W2, b2, Wroot, bconv):
    n = h.shape[0]
    nf = W0.shape[1]
    src = edge_index[0].astype(jnp.int32)
    dst = edge_index[1].astype(jnp.int32)

    hid = W1.shape[1]
    w2p = W2.reshape(hid, nf, nf).transpose(0, 2, 1).reshape(hid, nf * nf)
    w2p = w2p.astype(jnp.bfloat16)
    b2p = b2.reshape(nf, nf).T.reshape(nf * nf)
    gmat = jnp.repeat(jnp.eye(nf, dtype=jnp.bfloat16), nf, axis=0)

    deg2 = _sc_degree(dst, n, 16)
    out = _tc_node_proj(h, W0, b0)
    t = _tc_edge_mlp(edge_attr, Ws, bs, W1, b1)

    for _ in range(2):
        xg = _sc_gather(out, src)
        msg = _tc_msg(t, xg, w2p, b2p, gmat)
        agg2 = _sc_scatter_add(msg, dst, n)
        out = _tc_update(agg2, deg2, out, Wroot, bconv)
    return out
